# Initial kernel scaffold; baseline (speedup 1.0000x reference)
import jax
import jax.numpy as jnp
from jax.experimental import pallas as pl


def kernel(X, edge_index, edge_weight, c1w1, c1b1, c1w2, c1b2, c1w3, c1b3, chebW, chebB, c2w1, c2b1, c2w2, c2b2, c2w3, c2b3, bn_gamma, bn_beta):
    raise NotImplementedError("write your pallas kernel here")


# speedup vs baseline: 17.5316x; 17.5316x over previous
import functools

import jax
import jax.numpy as jnp
from jax import lax
from jax.experimental import pallas as pl
from jax.experimental.pallas import tpu as pltpu
from jax.experimental.pallas import tpu_sc as plsc

_N = 10000
_E = 320000
_C = 128
_CH = 64
_NS = 16
_NC = 2
_EPT = _E // _NS
_KE = 80
_NCH = _EPT // _KE
_NPT = _N // _NS
_NPAD = 632
_TMID = 10
_JOBS = _TMID // _NC

_MM_PREC = lax.Precision.HIGHEST
_SC_PARAMS = dict(
    compiler_params=pltpu.CompilerParams(
        needs_layout_passes=False, use_tc_tiling_on_sc=False),
)


def _mesh():
    return plsc.VectorSubcoreMesh(
        core_axis_name="c", subcore_axis_name="s",
        num_cores=_NC, num_subcores=_NS)


def _bcast_lane(v, e):
    idx = jnp.full((16, 1), e, dtype=jnp.int32)
    return lax.gather(
        v, idx,
        dimension_numbers=lax.GatherDimensionNumbers(
            offset_dims=(), collapsed_slice_dims=(0,), start_index_map=(0,)),
        slice_sizes=(1,),
        mode=lax.GatherScatterMode.PROMISE_IN_BOUNDS)


def _rsqrt16(x):
    i = lax.bitcast_convert_type(x, jnp.int32)
    i = jnp.full((16,), 0x5F3759DF, dtype=jnp.int32) - lax.shift_right_logical(i, 1)
    y = lax.bitcast_convert_type(i, jnp.float32)
    half = x * 0.5
    for _ in range(4):
        y = y * (1.5 - half * y * y)
    return y


def _norm_body(row_h, col_h, w_h, norm_h,
               row_v, col_v, w_v, weff_v, nout_v, dis_v, dloc_v,
               deg_sh, dis_sh):
    s = lax.axis_index("s")
    c = lax.axis_index("c")
    pltpu.sync_copy(row_h.at[s], row_v)
    pltpu.sync_copy(col_h.at[s], col_v)
    pltpu.sync_copy(w_h.at[s], w_v)

    zv = jnp.zeros((16,), jnp.float32)

    def zfill(i, carry):
        dloc_v[pl.ds(i * 16, 16)] = zv
        return carry
    lax.fori_loop(0, 40, zfill, 0)
    pltpu.sync_copy(dloc_v.at[pl.ds(0, _NPAD)],
                    deg_sh.at[pl.ds(s * _NPAD, _NPAD)])
    plsc.subcore_barrier()

    def wchunk(ch, carry):
        for g in range(_KE // 16):
            r16 = row_v[ch, pl.ds(g * 16, 16)]
            c16 = col_v[ch, pl.ds(g * 16, 16)]
            w16 = w_v[ch, pl.ds(g * 16, 16)]
            weff_v[ch, pl.ds(g * 16, 16)] = jnp.where(r16 == c16, 0.0, w16)
        pltpu.sync_copy(weff_v.at[ch], deg_sh.at[row_v.at[ch]], add=True)
        return carry
    lax.fori_loop(0, _NCH, wchunk, 0)
    plsc.subcore_barrier()

    pltpu.sync_copy(deg_sh.at[pl.ds(s * _NPAD, _NPAD)],
                    dloc_v.at[pl.ds(0, _NPAD)])

    def rchunk(i, carry):
        x = dloc_v[pl.ds(i * 16, 16)]
        y = jnp.where(x > 0.0, _rsqrt16(x), 0.0)
        dloc_v[pl.ds(i * 16, 16)] = y
        return carry
    lax.fori_loop(0, 40, rchunk, 0)
    pltpu.sync_copy(dloc_v.at[pl.ds(0, _NPAD)],
                    dis_sh.at[pl.ds(s * _NPAD, _NPAD)])
    plsc.subcore_barrier()

    pltpu.sync_copy(dis_sh, dis_v)

    def nchunk(ch, carry):
        for g in range(_KE // 16):
            r16 = row_v[ch, pl.ds(g * 16, 16)]
            c16 = col_v[ch, pl.ds(g * 16, 16)]
            we = weff_v[ch, pl.ds(g * 16, 16)]
            dr = plsc.load_gather(dis_v, [r16])
            dc = plsc.load_gather(dis_v, [c16])
            nout_v[ch, pl.ds(g * 16, 16)] = -(dr * we * dc)
        return carry
    lax.fori_loop(0, _NCH, nchunk, 0)

    @pl.when(c == 0)
    def _():
        pltpu.sync_copy(nout_v, norm_h.at[s])


def _norm_call(row3, col3, w3):
    f = pl.kernel(
        _norm_body,
        out_type=jax.ShapeDtypeStruct((_NS, _NCH, _KE), jnp.float32),
        mesh=_mesh(),
        scratch_types=[
            pltpu.VMEM((_NCH, _KE), jnp.int32),
            pltpu.VMEM((_NCH, _KE), jnp.int32),
            pltpu.VMEM((_NCH, _KE), jnp.float32),
            pltpu.VMEM((_NCH, _KE), jnp.float32),
            pltpu.VMEM((_NCH, _KE), jnp.float32),
            pltpu.VMEM((_NS * _NPAD,), jnp.float32),
            pltpu.VMEM((640,), jnp.float32),
            pltpu.VMEM_SHARED((_NS * _NPAD,), jnp.float32),
            pltpu.VMEM_SHARED((_NS * _NPAD,), jnp.float32),
        ],
        **_SC_PARAMS,
    )
    return f(row3, col3, w3)


def _level_body(z_h, row_h, col_h, norm_h, out_h,
                row_v, col_v, norm_v, gb0, gb1, zbuf_v,
                acc_sh, sem0, sem1):
    s = lax.axis_index("s")
    c = lax.axis_index("c")
    pltpu.sync_copy(row_h.at[s], row_v)
    pltpu.sync_copy(col_h.at[s], col_v)
    pltpu.sync_copy(norm_h.at[s], norm_v)

    zv = jnp.zeros((16,), jnp.float32)
    nvec = _CH // 16

    def zfill(i, carry):
        for f in range(nvec):
            zbuf_v[i, pl.ds(f * 16, 16)] = zv
        return carry
    lax.fori_loop(0, 125, zfill, 0)

    def shift(delta):
        def sbody(ch, carry):
            for g in range(_KE // 16):
                cur = row_v[ch, pl.ds(g * 16, 16)]
                row_v[ch, pl.ds(g * 16, 16)] = cur + delta
            return carry
        lax.fori_loop(0, _NCH, sbody, 0)
    shift(c * _N)

    def issue(ch, buf, sem):
        pltpu.async_copy(z_h.at[row_v.at[ch]], buf, sem)

    def waitbuf(buf, sem):
        pltpu.make_async_copy(z_h.at[pl.ds(0, _KE)], buf, sem).wait()

    def process(ch, buf):
        for g in range(_KE // 16):
            nv = norm_v[ch, pl.ds(g * 16, 16)]
            for e in range(16):
                bc = _bcast_lane(nv, e)
                r = g * 16 + e
                for f in range(nvec):
                    buf[r, pl.ds(f * 16, 16)] = buf[r, pl.ds(f * 16, 16)] * bc
        pltpu.sync_copy(buf, acc_sh.at[col_v.at[ch]], add=True)

    def job(ti, carry):
        t = c + _NC * ti
        for z5 in range(5):
            pltpu.sync_copy(zbuf_v, acc_sh.at[pl.ds(s * _NPT + z5 * 125, 125)])
        plsc.subcore_barrier()

        issue(0, gb0, sem0)

        def pair(j, inner):
            ch0 = 2 * j
            ch1 = 2 * j + 1
            issue(ch1, gb1, sem1)
            waitbuf(gb0, sem0)
            process(ch0, gb0)
            issue(jnp.minimum(ch0 + 2, _NCH - 1), gb0, sem0)
            waitbuf(gb1, sem1)
            process(ch1, gb1)
            return inner
        lax.fori_loop(0, _NCH // 2, pair, 0)
        waitbuf(gb0, sem0)

        plsc.subcore_barrier()
        pltpu.sync_copy(acc_sh.at[pl.ds(s * _NPT, _NPT)],
                        out_h.at[t, pl.ds(s * _NPT, _NPT)])
        shift(_NC * _N)
        return carry
    lax.fori_loop(0, _JOBS, job, 0)


def _level_call(zflat, row3, col3, norm3):
    f = pl.kernel(
        _level_body,
        out_type=jax.ShapeDtypeStruct((_TMID, _N, _CH), jnp.float32),
        mesh=_mesh(),
        scratch_types=[
            pltpu.VMEM((_NCH, _KE), jnp.int32),
            pltpu.VMEM((_NCH, _KE), jnp.int32),
            pltpu.VMEM((_NCH, _KE), jnp.float32),
            pltpu.VMEM((_KE, _CH), jnp.float32),
            pltpu.VMEM((_KE, _CH), jnp.float32),
            pltpu.VMEM((125, _CH), jnp.float32),
            pltpu.VMEM_SHARED((_N, _CH), jnp.float32),
            pltpu.SemaphoreType.DMA,
            pltpu.SemaphoreType.DMA,
        ],
        **_SC_PARAMS,
    )
    return f(zflat, row3, col3, norm3)


def _tconv1_call(x, W1, b1, W2, b2, W3, b3):
    nb = 400
    t_in, t_out = 12, 10

    def body(x_ref, w1_ref, b1_ref, w2_ref, b2_ref, w3_ref, b3_ref,
             o0_ref, o1_ref):
        x_blk = x_ref[...]
        xa = x_blk[0:t_out].reshape(t_out * nb, _C)
        xb = x_blk[1:t_out + 1].reshape(t_out * nb, _C)
        xc = x_blk[2:t_in].reshape(t_out * nb, _C)

        def tap(w_ref, b_ref):
            w = w_ref[...]
            y = jnp.dot(xa, w[0], preferred_element_type=jnp.float32,
                        precision=_MM_PREC)
            y = y + jnp.dot(xb, w[1], preferred_element_type=jnp.float32,
                            precision=_MM_PREC)
            y = y + jnp.dot(xc, w[2], preferred_element_type=jnp.float32,
                            precision=_MM_PREC)
            return y + b_ref[...][None, :]

        p = tap(w1_ref, b1_ref)
        q = tap(w2_ref, b2_ref)
        r = tap(w3_ref, b3_ref)
        h = jnp.maximum(p * jax.nn.sigmoid(q) + r, 0.0).reshape(t_out, nb, _C)
        o0_ref[...] = h[:, :, :_CH]
        o1_ref[...] = h[:, :, _CH:]

    wspec = pl.BlockSpec((3, _C, _C), lambda i: (0, 0, 0))
    bspec = pl.BlockSpec((_C,), lambda i: (0,))
    ospec = pl.BlockSpec((t_out, nb, _CH), lambda i: (0, i, 0))
    return pl.pallas_call(
        body,
        grid=(_N // nb,),
        compiler_params=pltpu.CompilerParams(
            vmem_limit_bytes=100 * 1024 * 1024),
        in_specs=[
            pl.BlockSpec((t_in, nb, _C), lambda i: (0, i, 0)),
            wspec, bspec, wspec, bspec, wspec, bspec,
        ],
        out_specs=(ospec, ospec),
        out_shape=(jax.ShapeDtypeStruct((t_out, _N, _CH), jnp.float32),
                   jax.ShapeDtypeStruct((t_out, _N, _CH), jnp.float32)),
    )(x, W1, b1, W2, b2, W3, b3)


def _combine_call(T0a, T0b, S1a, S1b, S2a, S2b, Wa, Wb, Wc, cb,
                  V1, vb1, V2, vb2, V3, vb3, gam, bet):
    nb = 400

    def body(t0a_ref, t0b_ref, s1a_ref, s1b_ref, s2a_ref, s2b_ref,
             wa_ref, wb_ref, wc_ref, cb_ref,
             v1_ref, b1_ref, v2_ref, b2_ref, v3_ref, b3_ref,
             g_ref, be_ref, o_ref):
        def halfmm(a_ref, b_ref_, w_ref):
            w = w_ref[...]
            ya = jnp.dot(a_ref[...].reshape(_TMID * nb, _CH), w[:_CH],
                         preferred_element_type=jnp.float32, precision=_MM_PREC)
            yb = jnp.dot(b_ref_[...].reshape(_TMID * nb, _CH), w[_CH:],
                         preferred_element_type=jnp.float32, precision=_MM_PREC)
            return ya + yb

        gm = halfmm(t0a_ref, t0b_ref, wa_ref)
        gm = gm + halfmm(s1a_ref, s1b_ref, wb_ref)
        gm = gm + halfmm(s2a_ref, s2b_ref, wc_ref)
        gm = jnp.maximum(gm + cb_ref[...][None, :], 0.0).reshape(_TMID, nb, _C)

        ga = gm[0:8].reshape(8 * nb, _C)
        gb = gm[1:9].reshape(8 * nb, _C)
        gc = gm[2:10].reshape(8 * nb, _C)

        def tap(v_ref, b_ref):
            v = v_ref[...]
            y = jnp.dot(ga, v[0], preferred_element_type=jnp.float32,
                        precision=_MM_PREC)
            y = y + jnp.dot(gb, v[1], preferred_element_type=jnp.float32,
                            precision=_MM_PREC)
            y = y + jnp.dot(gc, v[2], preferred_element_type=jnp.float32,
                            precision=_MM_PREC)
            return y + b_ref[...][None, :]

        p = tap(v1_ref, b1_ref)
        q = tap(v2_ref, b2_ref)
        r = tap(v3_ref, b3_ref)
        h = jnp.maximum(p * jax.nn.sigmoid(q) + r, 0.0).reshape(8, nb, _C)

        m = jnp.mean(h, axis=(0, 2))
        var = jnp.mean(jnp.square(h - m[None, :, None]), axis=(0, 2))
        scale = g_ref[...][:, 0] * lax.rsqrt(var + 1e-5)
        o_ref[...] = (h - m[None, :, None]) * scale[None, :, None] \
            + be_ref[...][:, 0][None, :, None]

    dspec = pl.BlockSpec((_TMID, nb, _CH), lambda i: (0, i, 0))
    wspec = pl.BlockSpec((_C, _C), lambda i: (0, 0))
    vspec = pl.BlockSpec((3, _C, _C), lambda i: (0, 0, 0))
    bspec = pl.BlockSpec((_C,), lambda i: (0,))
    nspec = pl.BlockSpec((nb, 1), lambda i: (i, 0))
    return pl.pallas_call(
        body,
        grid=(_N // nb,),
        compiler_params=pltpu.CompilerParams(
            vmem_limit_bytes=100 * 1024 * 1024),
        in_specs=[
            dspec, dspec, dspec, dspec, dspec, dspec,
            wspec, wspec, wspec, bspec,
            vspec, bspec, vspec, bspec, vspec, bspec,
            nspec, nspec,
        ],
        out_specs=pl.BlockSpec((8, nb, _C), lambda i: (0, i, 0)),
        out_shape=jax.ShapeDtypeStruct((8, _N, _C), jnp.float32),
    )(T0a, T0b, S1a, S1b, S2a, S2b, Wa, Wb, Wc, cb,
      V1, vb1, V2, vb2, V3, vb3,
      gam.reshape(_N, 1), bet.reshape(_N, 1))


def kernel(X, edge_index, edge_weight, c1w1, c1b1, c1w2, c1b2, c1w3, c1b3,
           chebW, chebB, c2w1, c2b1, c2w2, c2b2, c2w3, c2b3,
           bn_gamma, bn_beta):
    row3 = edge_index[0].reshape(_NS, _NCH, _KE)
    col3 = edge_index[1].reshape(_NS, _NCH, _KE)
    w3 = edge_weight.reshape(_NS, _NCH, _KE)
    norm3 = _norm_call(row3, col3, w3)

    W1 = jnp.transpose(c1w1[:, :, 0, :], (2, 1, 0))
    W2 = jnp.transpose(c1w2[:, :, 0, :], (2, 1, 0))
    W3 = jnp.transpose(c1w3[:, :, 0, :], (2, 1, 0))
    T0a, T0b = _tconv1_call(X[0], W1, c1b1, W2, c1b2, W3, c1b3)

    S1a = _level_call(T0a.reshape(_TMID * _N, _CH), row3, col3, norm3)
    S1b = _level_call(T0b.reshape(_TMID * _N, _CH), row3, col3, norm3)
    S2a = _level_call(S1a.reshape(_TMID * _N, _CH), row3, col3, norm3)
    S2b = _level_call(S1b.reshape(_TMID * _N, _CH), row3, col3, norm3)

    Wa = chebW[0] - chebW[2]
    Wb = chebW[1]
    Wc = 2.0 * chebW[2]
    V1 = jnp.transpose(c2w1[:, :, 0, :], (2, 1, 0))
    V2 = jnp.transpose(c2w2[:, :, 0, :], (2, 1, 0))
    V3 = jnp.transpose(c2w3[:, :, 0, :], (2, 1, 0))
    out = _combine_call(T0a, T0b, S1a, S1b, S2a, S2b, Wa, Wb, Wc, chebB,
                        V1, c2b1, V2, c2b2, V3, c2b3, bn_gamma, bn_beta)
    return out[None]


# speedup vs baseline: 19.6710x; 1.1220x over previous
import functools

import jax
import jax.numpy as jnp
from jax import lax
from jax.experimental import pallas as pl
from jax.experimental.pallas import tpu as pltpu
from jax.experimental.pallas import tpu_sc as plsc

_N = 10000
_E = 320000
_C = 128
_CH = 64
_NS = 16
_NC = 2
_EPT = _E // _NS
_KE = 80
_NCH = _EPT // _KE
_NPT = _N // _NS
_NPAD = 632
_TMID = 10
_JOBS = _TMID // _NC

_MM_PREC = lax.Precision.HIGHEST
_SC_PARAMS = dict(
    compiler_params=pltpu.CompilerParams(
        needs_layout_passes=False, use_tc_tiling_on_sc=False),
)


def _mesh():
    return plsc.VectorSubcoreMesh(
        core_axis_name="c", subcore_axis_name="s",
        num_cores=_NC, num_subcores=_NS)


def _bcast_lane(v, e):
    idx = jnp.full((16, 1), e, dtype=jnp.int32)
    return lax.gather(
        v, idx,
        dimension_numbers=lax.GatherDimensionNumbers(
            offset_dims=(), collapsed_slice_dims=(0,), start_index_map=(0,)),
        slice_sizes=(1,),
        mode=lax.GatherScatterMode.PROMISE_IN_BOUNDS)


def _rsqrt16(x):
    i = lax.bitcast_convert_type(x, jnp.int32)
    i = jnp.full((16,), 0x5F3759DF, dtype=jnp.int32) - lax.shift_right_logical(i, 1)
    y = lax.bitcast_convert_type(i, jnp.float32)
    half = x * 0.5
    for _ in range(4):
        y = y * (1.5 - half * y * y)
    return y


def _norm_body(row_h, col_h, w_h, norm_h,
               row_v, col_v, w_v, weff_v, nout_v, dis_v, dloc_v,
               deg_sh, dis_sh):
    s = lax.axis_index("s")
    c = lax.axis_index("c")
    pltpu.sync_copy(row_h.at[s], row_v)
    pltpu.sync_copy(col_h.at[s], col_v)
    pltpu.sync_copy(w_h.at[s], w_v)

    zv = jnp.zeros((16,), jnp.float32)

    def zfill(i, carry):
        dloc_v[pl.ds(i * 16, 16)] = zv
        return carry
    lax.fori_loop(0, 40, zfill, 0)
    pltpu.sync_copy(dloc_v.at[pl.ds(0, _NPAD)],
                    deg_sh.at[pl.ds(s * _NPAD, _NPAD)])
    plsc.subcore_barrier()

    def wchunk(ch, carry):
        for g in range(_KE // 16):
            r16 = row_v[ch, pl.ds(g * 16, 16)]
            c16 = col_v[ch, pl.ds(g * 16, 16)]
            w16 = w_v[ch, pl.ds(g * 16, 16)]
            weff_v[ch, pl.ds(g * 16, 16)] = jnp.where(r16 == c16, 0.0, w16)
        pltpu.sync_copy(weff_v.at[ch], deg_sh.at[row_v.at[ch]], add=True)
        return carry
    lax.fori_loop(0, _NCH, wchunk, 0)
    plsc.subcore_barrier()

    pltpu.sync_copy(deg_sh.at[pl.ds(s * _NPAD, _NPAD)],
                    dloc_v.at[pl.ds(0, _NPAD)])

    def rchunk(i, carry):
        x = dloc_v[pl.ds(i * 16, 16)]
        y = jnp.where(x > 0.0, _rsqrt16(x), 0.0)
        dloc_v[pl.ds(i * 16, 16)] = y
        return carry
    lax.fori_loop(0, 40, rchunk, 0)
    pltpu.sync_copy(dloc_v.at[pl.ds(0, _NPAD)],
                    dis_sh.at[pl.ds(s * _NPAD, _NPAD)])
    plsc.subcore_barrier()

    pltpu.sync_copy(dis_sh, dis_v)

    def nchunk(ch, carry):
        for g in range(_KE // 16):
            r16 = row_v[ch, pl.ds(g * 16, 16)]
            c16 = col_v[ch, pl.ds(g * 16, 16)]
            we = weff_v[ch, pl.ds(g * 16, 16)]
            dr = plsc.load_gather(dis_v, [r16])
            dc = plsc.load_gather(dis_v, [c16])
            nout_v[ch, pl.ds(g * 16, 16)] = -(dr * we * dc)
        return carry
    lax.fori_loop(0, _NCH, nchunk, 0)

    @pl.when(c == 0)
    def _():
        pltpu.sync_copy(nout_v, norm_h.at[s])


def _norm_call(row3, col3, w3):
    f = pl.kernel(
        _norm_body,
        out_type=jax.ShapeDtypeStruct((_NS, _NCH, _KE), jnp.float32),
        mesh=_mesh(),
        scratch_types=[
            pltpu.VMEM((_NCH, _KE), jnp.int32),
            pltpu.VMEM((_NCH, _KE), jnp.int32),
            pltpu.VMEM((_NCH, _KE), jnp.float32),
            pltpu.VMEM((_NCH, _KE), jnp.float32),
            pltpu.VMEM((_NCH, _KE), jnp.float32),
            pltpu.VMEM((_NS * _NPAD,), jnp.float32),
            pltpu.VMEM((640,), jnp.float32),
            pltpu.VMEM_SHARED((_NS * _NPAD,), jnp.float32),
            pltpu.VMEM_SHARED((_NS * _NPAD,), jnp.float32),
        ],
        **_SC_PARAMS,
    )
    return f(row3, col3, w3)


def _level_body(z_h, row_h, col_h, norm_h, out_h,
                row_v, col_v, norm_v, gb0, gb1, sb0, sb1, zbuf_v,
                acc_sh, gsem0, gsem1, ssem0, ssem1):
    s = lax.axis_index("s")
    c = lax.axis_index("c")
    pltpu.sync_copy(row_h.at[s], row_v)
    pltpu.sync_copy(col_h.at[s], col_v)
    pltpu.sync_copy(norm_h.at[s], norm_v)

    zv = jnp.zeros((16,), jnp.float32)
    nvec = _CH // 16

    def zfill(i, carry):
        for f in range(nvec):
            zbuf_v[i, pl.ds(f * 16, 16)] = zv
        return carry
    lax.fori_loop(0, 125, zfill, 0)

    def shift(delta):
        def sbody(ch, carry):
            for g in range(_KE // 16):
                cur = row_v[ch, pl.ds(g * 16, 16)]
                row_v[ch, pl.ds(g * 16, 16)] = cur + delta
            return carry
        lax.fori_loop(0, _NCH, sbody, 0)
    shift(c * _N)

    def issue_g(ch, buf, sem):
        pltpu.async_copy(z_h.at[row_v.at[ch]], buf, sem)

    def issue_s(ch, buf, sem):
        pltpu.async_copy(buf, acc_sh.at[col_v.at[ch]], sem, add=True)

    def waitbuf(buf, sem):
        pltpu.make_async_copy(z_h.at[pl.ds(0, _KE)], buf, sem).wait()

    def scale(ch, gbuf, sbuf):
        for g in range(_KE // 16):
            nv = norm_v[ch, pl.ds(g * 16, 16)]
            for e in range(16):
                bc = _bcast_lane(nv, e)
                r = g * 16 + e
                for f in range(nvec):
                    sbuf[r, pl.ds(f * 16, 16)] = gbuf[r, pl.ds(f * 16, 16)] * bc

    def zero_sbufs(i, carry):
        for f in range(nvec):
            sb0[i, pl.ds(f * 16, 16)] = zv
            sb1[i, pl.ds(f * 16, 16)] = zv
        return carry

    def job(ti, carry):
        t = c + _NC * ti
        lax.fori_loop(0, _KE, zero_sbufs, 0)
        for z5 in range(5):
            pltpu.sync_copy(zbuf_v, acc_sh.at[pl.ds(s * _NPT + z5 * 125, 125)])
        plsc.subcore_barrier()

        issue_s(0, sb0, ssem0)
        issue_s(0, sb1, ssem1)
        issue_g(0, gb0, gsem0)
        issue_g(1, gb1, gsem1)

        def pair(j, inner):
            ch0 = 2 * j
            ch1 = 2 * j + 1
            waitbuf(gb0, gsem0)
            waitbuf(sb0, ssem0)
            scale(ch0, gb0, sb0)
            issue_g(jnp.minimum(ch0 + 2, _NCH - 1), gb0, gsem0)
            issue_s(ch0, sb0, ssem0)
            waitbuf(gb1, gsem1)
            waitbuf(sb1, ssem1)
            scale(ch1, gb1, sb1)
            issue_g(jnp.minimum(ch1 + 2, _NCH - 1), gb1, gsem1)
            issue_s(ch1, sb1, ssem1)
            return inner
        lax.fori_loop(0, _NCH // 2, pair, 0)
        waitbuf(gb0, gsem0)
        waitbuf(gb1, gsem1)
        waitbuf(sb0, ssem0)
        waitbuf(sb1, ssem1)

        plsc.subcore_barrier()
        pltpu.sync_copy(acc_sh.at[pl.ds(s * _NPT, _NPT)],
                        out_h.at[t, pl.ds(s * _NPT, _NPT)])
        shift(_NC * _N)
        return carry
    lax.fori_loop(0, _JOBS, job, 0)


def _level_call(zflat, row3, col3, norm3):
    f = pl.kernel(
        _level_body,
        out_type=jax.ShapeDtypeStruct((_TMID, _N, _CH), jnp.float32),
        mesh=_mesh(),
        scratch_types=[
            pltpu.VMEM((_NCH, _KE), jnp.int32),
            pltpu.VMEM((_NCH, _KE), jnp.int32),
            pltpu.VMEM((_NCH, _KE), jnp.float32),
            pltpu.VMEM((_KE, _CH), jnp.float32),
            pltpu.VMEM((_KE, _CH), jnp.float32),
            pltpu.VMEM((_KE, _CH), jnp.float32),
            pltpu.VMEM((_KE, _CH), jnp.float32),
            pltpu.VMEM((125, _CH), jnp.float32),
            pltpu.VMEM_SHARED((_N, _CH), jnp.float32),
            pltpu.SemaphoreType.DMA,
            pltpu.SemaphoreType.DMA,
            pltpu.SemaphoreType.DMA,
            pltpu.SemaphoreType.DMA,
        ],
        **_SC_PARAMS,
    )
    return f(zflat, row3, col3, norm3)


def _tconv1_call(x, W1, b1, W2, b2, W3, b3):
    nb = 400
    t_in, t_out = 12, 10

    def body(x_ref, w1_ref, b1_ref, w2_ref, b2_ref, w3_ref, b3_ref,
             o0_ref, o1_ref):
        x_blk = x_ref[...]
        xa = x_blk[0:t_out].reshape(t_out * nb, _C)
        xb = x_blk[1:t_out + 1].reshape(t_out * nb, _C)
        xc = x_blk[2:t_in].reshape(t_out * nb, _C)

        def tap(w_ref, b_ref):
            w = w_ref[...]
            y = jnp.dot(xa, w[0], preferred_element_type=jnp.float32,
                        precision=_MM_PREC)
            y = y + jnp.dot(xb, w[1], preferred_element_type=jnp.float32,
                            precision=_MM_PREC)
            y = y + jnp.dot(xc, w[2], preferred_element_type=jnp.float32,
                            precision=_MM_PREC)
            return y + b_ref[...][None, :]

        p = tap(w1_ref, b1_ref)
        q = tap(w2_ref, b2_ref)
        r = tap(w3_ref, b3_ref)
        h = jnp.maximum(p * jax.nn.sigmoid(q) + r, 0.0).reshape(t_out, nb, _C)
        o0_ref[...] = h[:, :, :_CH]
        o1_ref[...] = h[:, :, _CH:]

    wspec = pl.BlockSpec((3, _C, _C), lambda i: (0, 0, 0))
    bspec = pl.BlockSpec((_C,), lambda i: (0,))
    ospec = pl.BlockSpec((t_out, nb, _CH), lambda i: (0, i, 0))
    return pl.pallas_call(
        body,
        grid=(_N // nb,),
        compiler_params=pltpu.CompilerParams(
            vmem_limit_bytes=100 * 1024 * 1024),
        in_specs=[
            pl.BlockSpec((t_in, nb, _C), lambda i: (0, i, 0)),
            wspec, bspec, wspec, bspec, wspec, bspec,
        ],
        out_specs=(ospec, ospec),
        out_shape=(jax.ShapeDtypeStruct((t_out, _N, _CH), jnp.float32),
                   jax.ShapeDtypeStruct((t_out, _N, _CH), jnp.float32)),
    )(x, W1, b1, W2, b2, W3, b3)


def _combine_call(T0a, T0b, S1a, S1b, S2a, S2b, Wa, Wb, Wc, cb,
                  V1, vb1, V2, vb2, V3, vb3, gam, bet):
    nb = 400

    def body(t0a_ref, t0b_ref, s1a_ref, s1b_ref, s2a_ref, s2b_ref,
             wa_ref, wb_ref, wc_ref, cb_ref,
             v1_ref, b1_ref, v2_ref, b2_ref, v3_ref, b3_ref,
             g_ref, be_ref, o_ref):
        def halfmm(a_ref, b_ref_, w_ref):
            w = w_ref[...]
            ya = jnp.dot(a_ref[...].reshape(_TMID * nb, _CH), w[:_CH],
                         preferred_element_type=jnp.float32, precision=_MM_PREC)
            yb = jnp.dot(b_ref_[...].reshape(_TMID * nb, _CH), w[_CH:],
                         preferred_element_type=jnp.float32, precision=_MM_PREC)
            return ya + yb

        gm = halfmm(t0a_ref, t0b_ref, wa_ref)
        gm = gm + halfmm(s1a_ref, s1b_ref, wb_ref)
        gm = gm + halfmm(s2a_ref, s2b_ref, wc_ref)
        gm = jnp.maximum(gm + cb_ref[...][None, :], 0.0).reshape(_TMID, nb, _C)

        ga = gm[0:8].reshape(8 * nb, _C)
        gb = gm[1:9].reshape(8 * nb, _C)
        gc = gm[2:10].reshape(8 * nb, _C)

        def tap(v_ref, b_ref):
            v = v_ref[...]
            y = jnp.dot(ga, v[0], preferred_element_type=jnp.float32,
                        precision=_MM_PREC)
            y = y + jnp.dot(gb, v[1], preferred_element_type=jnp.float32,
                            precision=_MM_PREC)
            y = y + jnp.dot(gc, v[2], preferred_element_type=jnp.float32,
                            precision=_MM_PREC)
            return y + b_ref[...][None, :]

        p = tap(v1_ref, b1_ref)
        q = tap(v2_ref, b2_ref)
        r = tap(v3_ref, b3_ref)
        h = jnp.maximum(p * jax.nn.sigmoid(q) + r, 0.0).reshape(8, nb, _C)

        m = jnp.mean(h, axis=(0, 2))
        var = jnp.mean(jnp.square(h - m[None, :, None]), axis=(0, 2))
        scale = g_ref[...][:, 0] * lax.rsqrt(var + 1e-5)
        o_ref[...] = (h - m[None, :, None]) * scale[None, :, None] \
            + be_ref[...][:, 0][None, :, None]

    dspec = pl.BlockSpec((_TMID, nb, _CH), lambda i: (0, i, 0))
    wspec = pl.BlockSpec((_C, _C), lambda i: (0, 0))
    vspec = pl.BlockSpec((3, _C, _C), lambda i: (0, 0, 0))
    bspec = pl.BlockSpec((_C,), lambda i: (0,))
    nspec = pl.BlockSpec((nb, 1), lambda i: (i, 0))
    return pl.pallas_call(
        body,
        grid=(_N // nb,),
        compiler_params=pltpu.CompilerParams(
            vmem_limit_bytes=100 * 1024 * 1024),
        in_specs=[
            dspec, dspec, dspec, dspec, dspec, dspec,
            wspec, wspec, wspec, bspec,
            vspec, bspec, vspec, bspec, vspec, bspec,
            nspec, nspec,
        ],
        out_specs=pl.BlockSpec((8, nb, _C), lambda i: (0, i, 0)),
        out_shape=jax.ShapeDtypeStruct((8, _N, _C), jnp.float32),
    )(T0a, T0b, S1a, S1b, S2a, S2b, Wa, Wb, Wc, cb,
      V1, vb1, V2, vb2, V3, vb3,
      gam.reshape(_N, 1), bet.reshape(_N, 1))


def kernel(X, edge_index, edge_weight, c1w1, c1b1, c1w2, c1b2, c1w3, c1b3,
           chebW, chebB, c2w1, c2b1, c2w2, c2b2, c2w3, c2b3,
           bn_gamma, bn_beta):
    row3 = edge_index[0].reshape(_NS, _NCH, _KE)
    col3 = edge_index[1].reshape(_NS, _NCH, _KE)
    w3 = edge_weight.reshape(_NS, _NCH, _KE)
    norm3 = _norm_call(row3, col3, w3)

    W1 = jnp.transpose(c1w1[:, :, 0, :], (2, 1, 0))
    W2 = jnp.transpose(c1w2[:, :, 0, :], (2, 1, 0))
    W3 = jnp.transpose(c1w3[:, :, 0, :], (2, 1, 0))
    T0a, T0b = _tconv1_call(X[0], W1, c1b1, W2, c1b2, W3, c1b3)

    S1a = _level_call(T0a.reshape(_TMID * _N, _CH), row3, col3, norm3)
    S1b = _level_call(T0b.reshape(_TMID * _N, _CH), row3, col3, norm3)
    S2a = _level_call(S1a.reshape(_TMID * _N, _CH), row3, col3, norm3)
    S2b = _level_call(S1b.reshape(_TMID * _N, _CH), row3, col3, norm3)

    Wa = chebW[0] - chebW[2]
    Wb = chebW[1]
    Wc = 2.0 * chebW[2]
    V1 = jnp.transpose(c2w1[:, :, 0, :], (2, 1, 0))
    V2 = jnp.transpose(c2w2[:, :, 0, :], (2, 1, 0))
    V3 = jnp.transpose(c2w3[:, :, 0, :], (2, 1, 0))
    out = _combine_call(T0a, T0b, S1a, S1b, S2a, S2b, Wa, Wb, Wc, chebB,
                        V1, c2b1, V2, c2b2, V3, c2b3, bn_gamma, bn_beta)
    return out[None]


# speedup vs baseline: 26.5660x; 1.3505x over previous
import functools

import jax
import jax.numpy as jnp
from jax import lax
from jax.experimental import pallas as pl
from jax.experimental.pallas import tpu as pltpu
from jax.experimental.pallas import tpu_sc as plsc

_N = 10000
_E = 320000
_C = 128
_CH = 64
_NS = 16
_NC = 2
_EPT = _E // _NS
_KE = 80
_NCH = _EPT // _KE
_NPT = _N // _NS
_NPAD = 632
_TMID = 10
_JOBS = _TMID // _NC

_MM_PREC = lax.Precision.DEFAULT
_SC_PARAMS = dict(
    compiler_params=pltpu.CompilerParams(
        needs_layout_passes=False, use_tc_tiling_on_sc=False),
)


def _mesh():
    return plsc.VectorSubcoreMesh(
        core_axis_name="c", subcore_axis_name="s",
        num_cores=_NC, num_subcores=_NS)


def _bcast_lane(v, e):
    idx = jnp.full((16, 1), e, dtype=jnp.int32)
    return lax.gather(
        v, idx,
        dimension_numbers=lax.GatherDimensionNumbers(
            offset_dims=(), collapsed_slice_dims=(0,), start_index_map=(0,)),
        slice_sizes=(1,),
        mode=lax.GatherScatterMode.PROMISE_IN_BOUNDS)


def _rsqrt16(x):
    i = lax.bitcast_convert_type(x, jnp.int32)
    i = jnp.full((16,), 0x5F3759DF, dtype=jnp.int32) - lax.shift_right_logical(i, 1)
    y = lax.bitcast_convert_type(i, jnp.float32)
    half = x * 0.5
    for _ in range(4):
        y = y * (1.5 - half * y * y)
    return y


def _norm_body(row_h, col_h, w_h, norm_h,
               row_v, col_v, w_v, weff_v, nout_v, dis_v, dloc_v,
               deg_sh, dis_sh):
    s = lax.axis_index("s")
    c = lax.axis_index("c")
    pltpu.sync_copy(row_h.at[s], row_v)
    pltpu.sync_copy(col_h.at[s], col_v)
    pltpu.sync_copy(w_h.at[s], w_v)

    zv = jnp.zeros((16,), jnp.float32)

    def zfill(i, carry):
        dloc_v[pl.ds(i * 16, 16)] = zv
        return carry
    lax.fori_loop(0, 40, zfill, 0)
    pltpu.sync_copy(dloc_v.at[pl.ds(0, _NPAD)],
                    deg_sh.at[pl.ds(s * _NPAD, _NPAD)])
    plsc.subcore_barrier()

    def wchunk(ch, carry):
        for g in range(_KE // 16):
            r16 = row_v[ch, pl.ds(g * 16, 16)]
            c16 = col_v[ch, pl.ds(g * 16, 16)]
            w16 = w_v[ch, pl.ds(g * 16, 16)]
            weff_v[ch, pl.ds(g * 16, 16)] = jnp.where(r16 == c16, 0.0, w16)
        pltpu.sync_copy(weff_v.at[ch], deg_sh.at[row_v.at[ch]], add=True)
        return carry
    lax.fori_loop(0, _NCH, wchunk, 0)
    plsc.subcore_barrier()

    pltpu.sync_copy(deg_sh.at[pl.ds(s * _NPAD, _NPAD)],
                    dloc_v.at[pl.ds(0, _NPAD)])

    def rchunk(i, carry):
        x = dloc_v[pl.ds(i * 16, 16)]
        y = jnp.where(x > 0.0, _rsqrt16(x), 0.0)
        dloc_v[pl.ds(i * 16, 16)] = y
        return carry
    lax.fori_loop(0, 40, rchunk, 0)
    pltpu.sync_copy(dloc_v.at[pl.ds(0, _NPAD)],
                    dis_sh.at[pl.ds(s * _NPAD, _NPAD)])
    plsc.subcore_barrier()

    pltpu.sync_copy(dis_sh, dis_v)

    def nchunk(ch, carry):
        for g in range(_KE // 16):
            r16 = row_v[ch, pl.ds(g * 16, 16)]
            c16 = col_v[ch, pl.ds(g * 16, 16)]
            we = weff_v[ch, pl.ds(g * 16, 16)]
            dr = plsc.load_gather(dis_v, [r16])
            dc = plsc.load_gather(dis_v, [c16])
            nout_v[ch, pl.ds(g * 16, 16)] = -(dr * we * dc)
        return carry
    lax.fori_loop(0, _NCH, nchunk, 0)

    @pl.when(c == 0)
    def _():
        pltpu.sync_copy(nout_v, norm_h.at[s])


def _norm_call(row3, col3, w3):
    f = pl.kernel(
        _norm_body,
        out_type=jax.ShapeDtypeStruct((_NS, _NCH, _KE), jnp.float32),
        mesh=_mesh(),
        scratch_types=[
            pltpu.VMEM((_NCH, _KE), jnp.int32),
            pltpu.VMEM((_NCH, _KE), jnp.int32),
            pltpu.VMEM((_NCH, _KE), jnp.float32),
            pltpu.VMEM((_NCH, _KE), jnp.float32),
            pltpu.VMEM((_NCH, _KE), jnp.float32),
            pltpu.VMEM((_NS * _NPAD,), jnp.float32),
            pltpu.VMEM((640,), jnp.float32),
            pltpu.VMEM_SHARED((_NS * _NPAD,), jnp.float32),
            pltpu.VMEM_SHARED((_NS * _NPAD,), jnp.float32),
        ],
        **_SC_PARAMS,
    )
    return f(row3, col3, w3)


def _level_body(z_h, row_h, col_h, norm_h, out_h,
                row_v, col_v, norm_v, gb0, gb1, sb0, sb1, zbuf_v,
                acc_sh, gsem0, gsem1, ssem0, ssem1):
    s = lax.axis_index("s")
    c = lax.axis_index("c")
    pltpu.sync_copy(row_h.at[s], row_v)
    pltpu.sync_copy(col_h.at[s], col_v)
    pltpu.sync_copy(norm_h.at[s], norm_v)

    zv = jnp.zeros((16,), jnp.float32)
    nvec = _CH // 16

    def zfill(i, carry):
        for f in range(nvec):
            zbuf_v[i, pl.ds(f * 16, 16)] = zv
        return carry
    lax.fori_loop(0, 125, zfill, 0)

    def shift(delta):
        def sbody(ch, carry):
            for g in range(_KE // 16):
                cur = row_v[ch, pl.ds(g * 16, 16)]
                row_v[ch, pl.ds(g * 16, 16)] = cur + delta
            return carry
        lax.fori_loop(0, _NCH, sbody, 0)
    shift(c * _N)

    def issue_g(ch, buf, sem):
        pltpu.async_copy(z_h.at[row_v.at[ch]], buf, sem)

    def issue_s(ch, buf, sem):
        pltpu.async_copy(buf, acc_sh.at[col_v.at[ch]], sem, add=True)

    def waitbuf(buf, sem):
        pltpu.make_async_copy(z_h.at[pl.ds(0, _KE)], buf, sem).wait()

    def scale(ch, gbuf, sbuf):
        for g in range(_KE // 16):
            nv = norm_v[ch, pl.ds(g * 16, 16)]
            for e in range(16):
                bc = _bcast_lane(nv, e)
                r = g * 16 + e
                for f in range(nvec):
                    sbuf[r, pl.ds(f * 16, 16)] = gbuf[r, pl.ds(f * 16, 16)] * bc

    def zero_sbufs(i, carry):
        for f in range(nvec):
            sb0[i, pl.ds(f * 16, 16)] = zv
            sb1[i, pl.ds(f * 16, 16)] = zv
        return carry

    def job(ti, carry):
        t = c + _NC * ti
        lax.fori_loop(0, _KE, zero_sbufs, 0)
        for z5 in range(5):
            pltpu.sync_copy(zbuf_v, acc_sh.at[pl.ds(s * _NPT + z5 * 125, 125)])
        plsc.subcore_barrier()

        issue_s(0, sb0, ssem0)
        issue_s(0, sb1, ssem1)
        issue_g(0, gb0, gsem0)
        issue_g(1, gb1, gsem1)

        def pair(j, inner):
            ch0 = 2 * j
            ch1 = 2 * j + 1
            waitbuf(gb0, gsem0)
            waitbuf(sb0, ssem0)
            scale(ch0, gb0, sb0)
            issue_g(jnp.minimum(ch0 + 2, _NCH - 1), gb0, gsem0)
            issue_s(ch0, sb0, ssem0)
            waitbuf(gb1, gsem1)
            waitbuf(sb1, ssem1)
            scale(ch1, gb1, sb1)
            issue_g(jnp.minimum(ch1 + 2, _NCH - 1), gb1, gsem1)
            issue_s(ch1, sb1, ssem1)
            return inner
        lax.fori_loop(0, _NCH // 2, pair, 0)
        waitbuf(gb0, gsem0)
        waitbuf(gb1, gsem1)
        waitbuf(sb0, ssem0)
        waitbuf(sb1, ssem1)

        plsc.subcore_barrier()
        pltpu.sync_copy(acc_sh.at[pl.ds(s * _NPT, _NPT)],
                        out_h.at[t, pl.ds(s * _NPT, _NPT)])
        shift(_NC * _N)
        return carry
    lax.fori_loop(0, _JOBS, job, 0)


def _level_call(zflat, row3, col3, norm3):
    f = pl.kernel(
        _level_body,
        out_type=jax.ShapeDtypeStruct((_TMID, _N, _CH), jnp.float32),
        mesh=_mesh(),
        scratch_types=[
            pltpu.VMEM((_NCH, _KE), jnp.int32),
            pltpu.VMEM((_NCH, _KE), jnp.int32),
            pltpu.VMEM((_NCH, _KE), jnp.float32),
            pltpu.VMEM((_KE, _CH), jnp.float32),
            pltpu.VMEM((_KE, _CH), jnp.float32),
            pltpu.VMEM((_KE, _CH), jnp.float32),
            pltpu.VMEM((_KE, _CH), jnp.float32),
            pltpu.VMEM((125, _CH), jnp.float32),
            pltpu.VMEM_SHARED((_N, _CH), jnp.float32),
            pltpu.SemaphoreType.DMA,
            pltpu.SemaphoreType.DMA,
            pltpu.SemaphoreType.DMA,
            pltpu.SemaphoreType.DMA,
        ],
        **_SC_PARAMS,
    )
    return f(zflat, row3, col3, norm3)


def _tconv1_call(x, W1, b1, W2, b2, W3, b3):
    nb = 400
    t_in, t_out = 12, 10

    def body(x_ref, w1_ref, b1_ref, w2_ref, b2_ref, w3_ref, b3_ref,
             o0_ref, o1_ref):
        x_blk = x_ref[...]
        xa = x_blk[0:t_out].reshape(t_out * nb, _C)
        xb = x_blk[1:t_out + 1].reshape(t_out * nb, _C)
        xc = x_blk[2:t_in].reshape(t_out * nb, _C)

        def tap(w_ref, b_ref):
            w = w_ref[...]
            y = jnp.dot(xa, w[0], preferred_element_type=jnp.float32,
                        precision=_MM_PREC)
            y = y + jnp.dot(xb, w[1], preferred_element_type=jnp.float32,
                            precision=_MM_PREC)
            y = y + jnp.dot(xc, w[2], preferred_element_type=jnp.float32,
                            precision=_MM_PREC)
            return y + b_ref[...][None, :]

        p = tap(w1_ref, b1_ref)
        q = tap(w2_ref, b2_ref)
        r = tap(w3_ref, b3_ref)
        h = jnp.maximum(p * jax.nn.sigmoid(q) + r, 0.0).reshape(t_out, nb, _C)
        o0_ref[...] = h[:, :, :_CH]
        o1_ref[...] = h[:, :, _CH:]

    wspec = pl.BlockSpec((3, _C, _C), lambda i: (0, 0, 0))
    bspec = pl.BlockSpec((_C,), lambda i: (0,))
    ospec = pl.BlockSpec((t_out, nb, _CH), lambda i: (0, i, 0))
    return pl.pallas_call(
        body,
        grid=(_N // nb,),
        compiler_params=pltpu.CompilerParams(
            vmem_limit_bytes=100 * 1024 * 1024),
        in_specs=[
            pl.BlockSpec((t_in, nb, _C), lambda i: (0, i, 0)),
            wspec, bspec, wspec, bspec, wspec, bspec,
        ],
        out_specs=(ospec, ospec),
        out_shape=(jax.ShapeDtypeStruct((t_out, _N, _CH), jnp.float32),
                   jax.ShapeDtypeStruct((t_out, _N, _CH), jnp.float32)),
    )(x, W1, b1, W2, b2, W3, b3)


def _combine_call(T0a, T0b, S1a, S1b, S2a, S2b, Wa, Wb, Wc, cb,
                  V1, vb1, V2, vb2, V3, vb3, gam, bet):
    nb = 400

    def body(t0a_ref, t0b_ref, s1a_ref, s1b_ref, s2a_ref, s2b_ref,
             wa_ref, wb_ref, wc_ref, cb_ref,
             v1_ref, b1_ref, v2_ref, b2_ref, v3_ref, b3_ref,
             g_ref, be_ref, o_ref):
        def halfmm(a_ref, b_ref_, w_ref):
            w = w_ref[...]
            ya = jnp.dot(a_ref[...].reshape(_TMID * nb, _CH), w[:_CH],
                         preferred_element_type=jnp.float32, precision=_MM_PREC)
            yb = jnp.dot(b_ref_[...].reshape(_TMID * nb, _CH), w[_CH:],
                         preferred_element_type=jnp.float32, precision=_MM_PREC)
            return ya + yb

        gm = halfmm(t0a_ref, t0b_ref, wa_ref)
        gm = gm + halfmm(s1a_ref, s1b_ref, wb_ref)
        gm = gm + halfmm(s2a_ref, s2b_ref, wc_ref)
        gm = jnp.maximum(gm + cb_ref[...][None, :], 0.0).reshape(_TMID, nb, _C)

        ga = gm[0:8].reshape(8 * nb, _C)
        gb = gm[1:9].reshape(8 * nb, _C)
        gc = gm[2:10].reshape(8 * nb, _C)

        def tap(v_ref, b_ref):
            v = v_ref[...]
            y = jnp.dot(ga, v[0], preferred_element_type=jnp.float32,
                        precision=_MM_PREC)
            y = y + jnp.dot(gb, v[1], preferred_element_type=jnp.float32,
                            precision=_MM_PREC)
            y = y + jnp.dot(gc, v[2], preferred_element_type=jnp.float32,
                            precision=_MM_PREC)
            return y + b_ref[...][None, :]

        p = tap(v1_ref, b1_ref)
        q = tap(v2_ref, b2_ref)
        r = tap(v3_ref, b3_ref)
        h = jnp.maximum(p * jax.nn.sigmoid(q) + r, 0.0).reshape(8, nb, _C)

        m = jnp.mean(h, axis=(0, 2))
        var = jnp.mean(jnp.square(h - m[None, :, None]), axis=(0, 2))
        scale = g_ref[...][:, 0] * lax.rsqrt(var + 1e-5)
        o_ref[...] = (h - m[None, :, None]) * scale[None, :, None] \
            + be_ref[...][:, 0][None, :, None]

    dspec = pl.BlockSpec((_TMID, nb, _CH), lambda i: (0, i, 0))
    wspec = pl.BlockSpec((_C, _C), lambda i: (0, 0))
    vspec = pl.BlockSpec((3, _C, _C), lambda i: (0, 0, 0))
    bspec = pl.BlockSpec((_C,), lambda i: (0,))
    nspec = pl.BlockSpec((nb, 1), lambda i: (i, 0))
    return pl.pallas_call(
        body,
        grid=(_N // nb,),
        compiler_params=pltpu.CompilerParams(
            vmem_limit_bytes=100 * 1024 * 1024),
        in_specs=[
            dspec, dspec, dspec, dspec, dspec, dspec,
            wspec, wspec, wspec, bspec,
            vspec, bspec, vspec, bspec, vspec, bspec,
            nspec, nspec,
        ],
        out_specs=pl.BlockSpec((8, nb, _C), lambda i: (0, i, 0)),
        out_shape=jax.ShapeDtypeStruct((8, _N, _C), jnp.float32),
    )(T0a, T0b, S1a, S1b, S2a, S2b, Wa, Wb, Wc, cb,
      V1, vb1, V2, vb2, V3, vb3,
      gam.reshape(_N, 1), bet.reshape(_N, 1))


def kernel(X, edge_index, edge_weight, c1w1, c1b1, c1w2, c1b2, c1w3, c1b3,
           chebW, chebB, c2w1, c2b1, c2w2, c2b2, c2w3, c2b3,
           bn_gamma, bn_beta):
    row3 = edge_index[0].reshape(_NS, _NCH, _KE)
    col3 = edge_index[1].reshape(_NS, _NCH, _KE)
    w3 = edge_weight.reshape(_NS, _NCH, _KE)
    norm3 = _norm_call(row3, col3, w3)

    W1 = jnp.transpose(c1w1[:, :, 0, :], (2, 1, 0))
    W2 = jnp.transpose(c1w2[:, :, 0, :], (2, 1, 0))
    W3 = jnp.transpose(c1w3[:, :, 0, :], (2, 1, 0))
    T0a, T0b = _tconv1_call(X[0], W1, c1b1, W2, c1b2, W3, c1b3)

    S1a = _level_call(T0a.reshape(_TMID * _N, _CH), row3, col3, norm3)
    S1b = _level_call(T0b.reshape(_TMID * _N, _CH), row3, col3, norm3)
    S2a = _level_call(S1a.reshape(_TMID * _N, _CH), row3, col3, norm3)
    S2b = _level_call(S1b.reshape(_TMID * _N, _CH), row3, col3, norm3)

    Wa = chebW[0] - chebW[2]
    Wb = chebW[1]
    Wc = 2.0 * chebW[2]
    V1 = jnp.transpose(c2w1[:, :, 0, :], (2, 1, 0))
    V2 = jnp.transpose(c2w2[:, :, 0, :], (2, 1, 0))
    V3 = jnp.transpose(c2w3[:, :, 0, :], (2, 1, 0))
    out = _combine_call(T0a, T0b, S1a, S1b, S2a, S2b, Wa, Wb, Wc, chebB,
                        V1, c2b1, V2, c2b2, V3, c2b3, bn_gamma, bn_beta)
    return out[None]


# speedup vs baseline: 27.0652x; 1.0188x over previous
import functools

import jax
import jax.numpy as jnp
from jax import lax
from jax.experimental import pallas as pl
from jax.experimental.pallas import tpu as pltpu
from jax.experimental.pallas import tpu_sc as plsc

_N = 10000
_E = 320000
_C = 128
_CH = 64
_NS = 16
_NC = 2
_EPT = _E // _NS
_KE = 80
_NCH = _EPT // _KE
_NPT = _N // _NS
_NPAD = 632
_TMID = 10
_JOBS = _TMID // _NC

_MM_PREC = lax.Precision.DEFAULT
_SC_PARAMS = dict(
    compiler_params=pltpu.CompilerParams(
        needs_layout_passes=False, use_tc_tiling_on_sc=False),
)


def _mesh():
    return plsc.VectorSubcoreMesh(
        core_axis_name="c", subcore_axis_name="s",
        num_cores=_NC, num_subcores=_NS)


def _bcast_lane(v, e):
    idx = jnp.full((16, 1), e, dtype=jnp.int32)
    return lax.gather(
        v, idx,
        dimension_numbers=lax.GatherDimensionNumbers(
            offset_dims=(), collapsed_slice_dims=(0,), start_index_map=(0,)),
        slice_sizes=(1,),
        mode=lax.GatherScatterMode.PROMISE_IN_BOUNDS)


def _rsqrt16(x):
    i = lax.bitcast_convert_type(x, jnp.int32)
    i = jnp.full((16,), 0x5F3759DF, dtype=jnp.int32) - lax.shift_right_logical(i, 1)
    y = lax.bitcast_convert_type(i, jnp.float32)
    half = x * 0.5
    for _ in range(4):
        y = y * (1.5 - half * y * y)
    return y


def _norm_body(row_h, col_h, w_h, norm_h,
               row_v, col_v, w_v, weff_v, nout_v, dis_v, dloc_v,
               deg_sh, dis_sh):
    s = lax.axis_index("s")
    c = lax.axis_index("c")
    pltpu.sync_copy(row_h.at[s], row_v)
    pltpu.sync_copy(col_h.at[s], col_v)
    pltpu.sync_copy(w_h.at[s], w_v)

    zv = jnp.zeros((16,), jnp.float32)

    def zfill(i, carry):
        dloc_v[pl.ds(i * 16, 16)] = zv
        return carry
    lax.fori_loop(0, 40, zfill, 0)
    pltpu.sync_copy(dloc_v.at[pl.ds(0, _NPAD)],
                    deg_sh.at[pl.ds(s * _NPAD, _NPAD)])
    plsc.subcore_barrier()

    def wchunk(ch, carry):
        for g in range(_KE // 16):
            r16 = row_v[ch, pl.ds(g * 16, 16)]
            c16 = col_v[ch, pl.ds(g * 16, 16)]
            w16 = w_v[ch, pl.ds(g * 16, 16)]
            weff_v[ch, pl.ds(g * 16, 16)] = jnp.where(r16 == c16, 0.0, w16)
        pltpu.sync_copy(weff_v.at[ch], deg_sh.at[row_v.at[ch]], add=True)
        return carry
    lax.fori_loop(0, _NCH, wchunk, 0)
    plsc.subcore_barrier()

    pltpu.sync_copy(deg_sh.at[pl.ds(s * _NPAD, _NPAD)],
                    dloc_v.at[pl.ds(0, _NPAD)])

    def rchunk(i, carry):
        x = dloc_v[pl.ds(i * 16, 16)]
        y = jnp.where(x > 0.0, _rsqrt16(x), 0.0)
        dloc_v[pl.ds(i * 16, 16)] = y
        return carry
    lax.fori_loop(0, 40, rchunk, 0)
    pltpu.sync_copy(dloc_v.at[pl.ds(0, _NPAD)],
                    dis_sh.at[pl.ds(s * _NPAD, _NPAD)])
    plsc.subcore_barrier()

    pltpu.sync_copy(dis_sh, dis_v)

    def nchunk(ch, carry):
        for g in range(_KE // 16):
            r16 = row_v[ch, pl.ds(g * 16, 16)]
            c16 = col_v[ch, pl.ds(g * 16, 16)]
            we = weff_v[ch, pl.ds(g * 16, 16)]
            dr = plsc.load_gather(dis_v, [r16])
            dc = plsc.load_gather(dis_v, [c16])
            nout_v[ch, pl.ds(g * 16, 16)] = -(dr * we * dc)
        return carry
    lax.fori_loop(0, _NCH, nchunk, 0)

    @pl.when(c == 0)
    def _():
        pltpu.sync_copy(nout_v, norm_h.at[s])


def _norm_call(row3, col3, w3):
    f = pl.kernel(
        _norm_body,
        out_type=jax.ShapeDtypeStruct((_NS, _NCH, _KE), jnp.float32),
        mesh=_mesh(),
        scratch_types=[
            pltpu.VMEM((_NCH, _KE), jnp.int32),
            pltpu.VMEM((_NCH, _KE), jnp.int32),
            pltpu.VMEM((_NCH, _KE), jnp.float32),
            pltpu.VMEM((_NCH, _KE), jnp.float32),
            pltpu.VMEM((_NCH, _KE), jnp.float32),
            pltpu.VMEM((_NS * _NPAD,), jnp.float32),
            pltpu.VMEM((640,), jnp.float32),
            pltpu.VMEM_SHARED((_NS * _NPAD,), jnp.float32),
            pltpu.VMEM_SHARED((_NS * _NPAD,), jnp.float32),
        ],
        **_SC_PARAMS,
    )
    return f(row3, col3, w3)


def _level_body(z_h, row_h, col_h, norm_h, out_h,
                row_v, col_v, norm_v, gb0, gb1, sb0, sb1, zbuf_v, wb_v,
                acc_sh, gsem0, gsem1, ssem0, ssem1):
    s = lax.axis_index("s")
    c = lax.axis_index("c")
    pltpu.sync_copy(row_h.at[s], row_v)
    pltpu.sync_copy(col_h.at[s], col_v)
    pltpu.sync_copy(norm_h.at[s], norm_v)

    zv = jnp.zeros((16,), jnp.float32)
    nvec = _CH // 16

    def zfill(i, carry):
        for f in range(nvec):
            zbuf_v[i, pl.ds(f * 16, 16)] = zv
        return carry

    def shift(delta):
        def sbody(ch, carry):
            for g in range(_KE // 16):
                cur = row_v[ch, pl.ds(g * 16, 16)]
                row_v[ch, pl.ds(g * 16, 16)] = cur + delta
            return carry
        lax.fori_loop(0, _NCH, sbody, 0)
    shift(c * _N)

    def issue_g(ch, buf, sem):
        pltpu.async_copy(z_h.at[row_v.at[ch]], buf, sem)

    def issue_s(ch, buf, sem):
        pltpu.async_copy(buf, acc_sh.at[col_v.at[ch]], sem, add=True)

    def waitbuf(buf, sem):
        pltpu.make_async_copy(z_h.at[pl.ds(0, _KE)], buf, sem).wait()

    def scale(ch, gbuf, sbuf):
        for g in range(_KE // 16):
            nv = norm_v[ch, pl.ds(g * 16, 16)]
            for e in range(16):
                bc = _bcast_lane(nv, e)
                r = g * 16 + e
                for h2 in range(_CH // 32):
                    ab = gbuf[r, pl.ds(h2 * 32, 32)]
                    a, b = plsc.unpack(ab, format=plsc.PackFormat.INTERLEAVED)
                    sbuf[r, pl.ds(h2 * 32, 16)] = a * bc
                    sbuf[r, pl.ds(h2 * 32 + 16, 16)] = b * bc

    def zero_sbufs(i, carry):
        for f in range(nvec):
            sb0[i, pl.ds(f * 16, 16)] = zv
            sb1[i, pl.ds(f * 16, 16)] = zv
        return carry

    def job(ti, carry):
        t = c + _NC * ti
        lax.fori_loop(0, _KE, zero_sbufs, 0)
        lax.fori_loop(0, 125, zfill, 0)
        for z5 in range(5):
            pltpu.sync_copy(zbuf_v, acc_sh.at[pl.ds(s * _NPT + z5 * 125, 125)])
        plsc.subcore_barrier()

        issue_s(0, sb0, ssem0)
        issue_s(0, sb1, ssem1)
        issue_g(0, gb0, gsem0)
        issue_g(1, gb1, gsem1)

        def pair(j, inner):
            ch0 = 2 * j
            ch1 = 2 * j + 1
            waitbuf(gb0, gsem0)
            waitbuf(sb0, ssem0)
            scale(ch0, gb0, sb0)
            issue_g(jnp.minimum(ch0 + 2, _NCH - 1), gb0, gsem0)
            issue_s(ch0, sb0, ssem0)
            waitbuf(gb1, gsem1)
            waitbuf(sb1, ssem1)
            scale(ch1, gb1, sb1)
            issue_g(jnp.minimum(ch1 + 2, _NCH - 1), gb1, gsem1)
            issue_s(ch1, sb1, ssem1)
            return inner
        lax.fori_loop(0, _NCH // 2, pair, 0)
        waitbuf(gb0, gsem0)
        waitbuf(gb1, gsem1)
        waitbuf(sb0, ssem0)
        waitbuf(sb1, ssem1)

        plsc.subcore_barrier()
        for z5 in range(5):
            pltpu.sync_copy(acc_sh.at[pl.ds(s * _NPT + z5 * 125, 125)], zbuf_v)

            def cvt(i, carry2):
                for h2 in range(_CH // 32):
                    a = zbuf_v[i, pl.ds(h2 * 32, 16)]
                    b = zbuf_v[i, pl.ds(h2 * 32 + 16, 16)]
                    wb_v[i, pl.ds(h2 * 32, 32)] = plsc.pack(
                        a, b, format=plsc.PackFormat.INTERLEAVED)
                return carry2
            lax.fori_loop(0, 125, cvt, 0)
            pltpu.sync_copy(wb_v,
                            out_h.at[t, pl.ds(s * _NPT + z5 * 125, 125)])
        shift(_NC * _N)
        return carry
    lax.fori_loop(0, _JOBS, job, 0)


def _level_call(zflat, row3, col3, norm3):
    f = pl.kernel(
        _level_body,
        out_type=jax.ShapeDtypeStruct((_TMID, _N, _CH), jnp.bfloat16),
        mesh=_mesh(),
        scratch_types=[
            pltpu.VMEM((_NCH, _KE), jnp.int32),
            pltpu.VMEM((_NCH, _KE), jnp.int32),
            pltpu.VMEM((_NCH, _KE), jnp.float32),
            pltpu.VMEM((_KE, _CH), jnp.bfloat16),
            pltpu.VMEM((_KE, _CH), jnp.bfloat16),
            pltpu.VMEM((_KE, _CH), jnp.float32),
            pltpu.VMEM((_KE, _CH), jnp.float32),
            pltpu.VMEM((125, _CH), jnp.float32),
            pltpu.VMEM((125, _CH), jnp.bfloat16),
            pltpu.VMEM_SHARED((_N, _CH), jnp.float32),
            pltpu.SemaphoreType.DMA,
            pltpu.SemaphoreType.DMA,
            pltpu.SemaphoreType.DMA,
            pltpu.SemaphoreType.DMA,
        ],
        **_SC_PARAMS,
    )
    return f(zflat, row3, col3, norm3)


def _tconv1_call(x, W1, b1, W2, b2, W3, b3):
    nb = 400
    t_in, t_out = 12, 10

    def body(x_ref, w1_ref, b1_ref, w2_ref, b2_ref, w3_ref, b3_ref,
             o0_ref, o1_ref):
        x_blk = x_ref[...]
        xa = x_blk[0:t_out].reshape(t_out * nb, _C)
        xb = x_blk[1:t_out + 1].reshape(t_out * nb, _C)
        xc = x_blk[2:t_in].reshape(t_out * nb, _C)

        def tap(w_ref, b_ref):
            w = w_ref[...]
            y = jnp.dot(xa, w[0], preferred_element_type=jnp.float32,
                        precision=_MM_PREC)
            y = y + jnp.dot(xb, w[1], preferred_element_type=jnp.float32,
                            precision=_MM_PREC)
            y = y + jnp.dot(xc, w[2], preferred_element_type=jnp.float32,
                            precision=_MM_PREC)
            return y + b_ref[...][None, :]

        p = tap(w1_ref, b1_ref)
        q = tap(w2_ref, b2_ref)
        r = tap(w3_ref, b3_ref)
        h = jnp.maximum(p * jax.nn.sigmoid(q) + r, 0.0).reshape(t_out, nb, _C)
        o0_ref[...] = h[:, :, :_CH].astype(jnp.bfloat16)
        o1_ref[...] = h[:, :, _CH:].astype(jnp.bfloat16)

    wspec = pl.BlockSpec((3, _C, _C), lambda i: (0, 0, 0))
    bspec = pl.BlockSpec((_C,), lambda i: (0,))
    ospec = pl.BlockSpec((t_out, nb, _CH), lambda i: (0, i, 0))
    return pl.pallas_call(
        body,
        grid=(_N // nb,),
        compiler_params=pltpu.CompilerParams(
            vmem_limit_bytes=100 * 1024 * 1024),
        in_specs=[
            pl.BlockSpec((t_in, nb, _C), lambda i: (0, i, 0)),
            wspec, bspec, wspec, bspec, wspec, bspec,
        ],
        out_specs=(ospec, ospec),
        out_shape=(jax.ShapeDtypeStruct((t_out, _N, _CH), jnp.bfloat16),
                   jax.ShapeDtypeStruct((t_out, _N, _CH), jnp.bfloat16)),
    )(x, W1, b1, W2, b2, W3, b3)


def _combine_call(T0a, T0b, S1a, S1b, S2a, S2b, Wa, Wb, Wc, cb,
                  V1, vb1, V2, vb2, V3, vb3, gam, bet):
    nb = 400

    def body(t0a_ref, t0b_ref, s1a_ref, s1b_ref, s2a_ref, s2b_ref,
             wa_ref, wb_ref, wc_ref, cb_ref,
             v1_ref, b1_ref, v2_ref, b2_ref, v3_ref, b3_ref,
             g_ref, be_ref, o_ref):
        def halfmm(a_ref, b_ref_, w_ref):
            w = w_ref[...]
            ya = jnp.dot(a_ref[...].reshape(_TMID * nb, _CH), w[:_CH],
                         preferred_element_type=jnp.float32, precision=_MM_PREC)
            yb = jnp.dot(b_ref_[...].reshape(_TMID * nb, _CH), w[_CH:],
                         preferred_element_type=jnp.float32, precision=_MM_PREC)
            return ya + yb

        gm = halfmm(t0a_ref, t0b_ref, wa_ref)
        gm = gm + halfmm(s1a_ref, s1b_ref, wb_ref)
        gm = gm + halfmm(s2a_ref, s2b_ref, wc_ref)
        gm = jnp.maximum(gm + cb_ref[...][None, :], 0.0).reshape(_TMID, nb, _C)

        ga = gm[0:8].reshape(8 * nb, _C)
        gb = gm[1:9].reshape(8 * nb, _C)
        gc = gm[2:10].reshape(8 * nb, _C)

        def tap(v_ref, b_ref):
            v = v_ref[...]
            y = jnp.dot(ga, v[0], preferred_element_type=jnp.float32,
                        precision=_MM_PREC)
            y = y + jnp.dot(gb, v[1], preferred_element_type=jnp.float32,
                            precision=_MM_PREC)
            y = y + jnp.dot(gc, v[2], preferred_element_type=jnp.float32,
                            precision=_MM_PREC)
            return y + b_ref[...][None, :]

        p = tap(v1_ref, b1_ref)
        q = tap(v2_ref, b2_ref)
        r = tap(v3_ref, b3_ref)
        h = jnp.maximum(p * jax.nn.sigmoid(q) + r, 0.0).reshape(8, nb, _C)

        m = jnp.mean(h, axis=(0, 2))
        var = jnp.mean(jnp.square(h - m[None, :, None]), axis=(0, 2))
        scale = g_ref[...][:, 0] * lax.rsqrt(var + 1e-5)
        o_ref[...] = (h - m[None, :, None]) * scale[None, :, None] \
            + be_ref[...][:, 0][None, :, None]

    dspec = pl.BlockSpec((_TMID, nb, _CH), lambda i: (0, i, 0))
    wspec = pl.BlockSpec((_C, _C), lambda i: (0, 0))
    vspec = pl.BlockSpec((3, _C, _C), lambda i: (0, 0, 0))
    bspec = pl.BlockSpec((_C,), lambda i: (0,))
    nspec = pl.BlockSpec((nb, 1), lambda i: (i, 0))
    return pl.pallas_call(
        body,
        grid=(_N // nb,),
        compiler_params=pltpu.CompilerParams(
            vmem_limit_bytes=100 * 1024 * 1024),
        in_specs=[
            dspec, dspec, dspec, dspec, dspec, dspec,
            wspec, wspec, wspec, bspec,
            vspec, bspec, vspec, bspec, vspec, bspec,
            nspec, nspec,
        ],
        out_specs=pl.BlockSpec((8, nb, _C), lambda i: (0, i, 0)),
        out_shape=jax.ShapeDtypeStruct((8, _N, _C), jnp.float32),
    )(T0a, T0b, S1a, S1b, S2a, S2b, Wa, Wb, Wc, cb,
      V1, vb1, V2, vb2, V3, vb3,
      gam.reshape(_N, 1), bet.reshape(_N, 1))


def kernel(X, edge_index, edge_weight, c1w1, c1b1, c1w2, c1b2, c1w3, c1b3,
           chebW, chebB, c2w1, c2b1, c2w2, c2b2, c2w3, c2b3,
           bn_gamma, bn_beta):
    row3 = edge_index[0].reshape(_NS, _NCH, _KE)
    col3 = edge_index[1].reshape(_NS, _NCH, _KE)
    w3 = edge_weight.reshape(_NS, _NCH, _KE)
    norm3 = _norm_call(row3, col3, w3)

    W1 = jnp.transpose(c1w1[:, :, 0, :], (2, 1, 0))
    W2 = jnp.transpose(c1w2[:, :, 0, :], (2, 1, 0))
    W3 = jnp.transpose(c1w3[:, :, 0, :], (2, 1, 0))
    T0a, T0b = _tconv1_call(X[0], W1, c1b1, W2, c1b2, W3, c1b3)

    S1a = _level_call(T0a.reshape(_TMID * _N, _CH), row3, col3, norm3)
    S1b = _level_call(T0b.reshape(_TMID * _N, _CH), row3, col3, norm3)
    S2a = _level_call(S1a.reshape(_TMID * _N, _CH), row3, col3, norm3)
    S2b = _level_call(S1b.reshape(_TMID * _N, _CH), row3, col3, norm3)

    Wa = chebW[0] - chebW[2]
    Wb = chebW[1]
    Wc = 2.0 * chebW[2]
    V1 = jnp.transpose(c2w1[:, :, 0, :], (2, 1, 0))
    V2 = jnp.transpose(c2w2[:, :, 0, :], (2, 1, 0))
    V3 = jnp.transpose(c2w3[:, :, 0, :], (2, 1, 0))
    out = _combine_call(T0a, T0b, S1a, S1b, S2a, S2b, Wa, Wb, Wc, chebB,
                        V1, c2b1, V2, c2b2, V3, c2b3, bn_gamma, bn_beta)
    return out[None]
